# baseline (speedup 1.0000x reference)
import functools

import jax
import jax.numpy as jnp
from jax import lax
from jax.experimental import pallas as pl
from jax.experimental.pallas import tpu as pltpu
from jax.experimental.pallas import tpu_sc as plsc

NUM_VARS = 100000
IN_DIM = 128
OUT_DIM = 128
NUM_CLUSTERS = 2000
VARS_PER_CLUSTER = 50
NUM_CEDGES = 32000
SHARED_PER_EDGE = 8
NEG_SLOPE = 0.2

NC = 2
NS = 16
NW = NC * NS

C_PAD = 2048
CPW = C_PAD // NW
VPC_PAD = 56
IDX_ROW = CPW * VPC_PAD

E_PW = NUM_CEDGES // NW
E_CHUNKS = [(0, 128), (128, 128), (256, 128), (384, 128),
            (512, 128), (640, 128), (768, 128), (896, 104)]

S_PT = NUM_CEDGES // NS * SHARED_PER_EDGE
CH = 10000
NCHUNK = NUM_VARS // (NC * CH)
R_PT = 624
R_TAIL = CH - NS * R_PT
BATCH = 32
RPB = BATCH // 16

_mesh = plsc.VectorSubcoreMesh(
    core_axis_name="c", subcore_axis_name="s", num_cores=NC, num_subcores=NS)

_GDN = lax.GatherDimensionNumbers(
    offset_dims=(), collapsed_slice_dims=(0,), start_index_map=(0,))


def _rot(v, sh):
    idx = ((lax.iota(jnp.int32, 16) + sh) & 15)[:, None]
    return lax.gather(v, idx, _GDN, (1,),
                      mode=lax.GatherScatterMode.PROMISE_IN_BOUNDS)


def _allreduce(v, op):
    for sh in (8, 4, 2, 1):
        v = op(v, _rot(v, sh))
    return v


G2 = 2 * VPC_PAD
NPAIR = CPW // 2
DEPTH1 = 4


@functools.partial(
    pl.kernel,
    out_type=jax.ShapeDtypeStruct((C_PAD, IN_DIM), jnp.float32),
    mesh=_mesh,
    scratch_types=[
        pltpu.VMEM((IDX_ROW + DEPTH1 * G2,), jnp.int32),
        [pltpu.VMEM((G2, IN_DIM), jnp.float32) for _ in range(DEPTH1)],
        pltpu.VMEM((CPW, IN_DIM), jnp.float32),
        [pltpu.SemaphoreType.DMA for _ in range(DEPTH1)],
    ],
    compiler_params=pltpu.CompilerParams(needs_layout_passes=False),
)
def _sc_cluster_means(x_hbm, cvi_hbm, cf_hbm, idxb, rows, cfb, sems):
    w = lax.axis_index("s") * NC + lax.axis_index("c")
    pltpu.sync_copy(cvi_hbm.at[pl.ds(w * IDX_ROW, IDX_ROW)],
                    idxb.at[pl.ds(0, IDX_ROW)])
    for v in range(DEPTH1 * G2 // 16):
        idxb[pl.ds(IDX_ROW + v * 16, 16)] = jnp.zeros((16,), jnp.int32)

    def fire(p, j):
        pltpu.async_copy(x_hbm.at[idxb.at[pl.ds(p * G2, G2)]], rows[j],
                         sems[j])

    def wait(j):
        pltpu.make_async_copy(x_hbm.at[idxb.at[pl.ds(0, G2)]], rows[j],
                              sems[j]).wait()

    def reduce_pair(p, j):
        inv = jnp.float32(1.0 / VARS_PER_CLUSTER)
        for i in range(2):
            def racc(r, accs):
                return tuple(
                    accs[d] + rows[j][i * VPC_PAD + r, pl.ds(d * 16, 16)]
                    for d in range(IN_DIM // 16))
            accs = lax.fori_loop(
                0, VARS_PER_CLUSTER, racc,
                tuple(jnp.zeros((16,), jnp.float32)
                      for _ in range(IN_DIM // 16)))
            for d in range(IN_DIM // 16):
                cfb[2 * p + i, pl.ds(d * 16, 16)] = accs[d] * inv

    for j in range(DEPTH1):
        fire(j, j)

    def body(g, _):
        for j in range(DEPTH1):
            wait(j)
            reduce_pair(DEPTH1 * g + j, j)
            fire(DEPTH1 * g + DEPTH1 + j, j)
        return 0

    lax.fori_loop(0, NPAIR // DEPTH1, body, 0)
    for j in range(DEPTH1):
        wait(j)
    pltpu.sync_copy(cfb, cf_hbm.at[pl.ds(w * CPW, CPW)])


def _tc_qkv_body(cf_ref, wq_ref, wk_ref, wv_ref, hw_ref, q_ref, k_ref, v_ref):
    cf = cf_ref[...]
    dn = (((1,), (1,)), ((), ()))
    scale = jnp.float32(1.0) / jnp.sqrt(jnp.float32(OUT_DIM))
    q_ref[...] = lax.dot_general(
        cf, wq_ref[...], dn, preferred_element_type=jnp.float32) * scale
    k_ref[...] = lax.dot_general(
        cf, wk_ref[...], dn, preferred_element_type=jnp.float32)
    hw = jnp.sum(hw_ref[...]) * jnp.float32(1.0 / 4.0)
    v_ref[...] = lax.dot_general(
        cf, wv_ref[...], dn, preferred_element_type=jnp.float32) * hw


_tc_qkv = pl.pallas_call(
    _tc_qkv_body,
    out_shape=[jax.ShapeDtypeStruct((C_PAD, OUT_DIM), jnp.float32)] * 3,
)


@functools.partial(
    pl.kernel,
    out_type=jax.ShapeDtypeStruct((NUM_CEDGES, OUT_DIM), jnp.float32),
    mesh=_mesh,
    scratch_types=[
        pltpu.VMEM((E_PW,), jnp.int32),
        pltpu.VMEM((E_PW,), jnp.int32),
        [pltpu.VMEM((128, OUT_DIM), jnp.float32) for _ in range(6)],
        [pltpu.SemaphoreType.DMA for _ in range(6)],
    ],
    compiler_params=pltpu.CompilerParams(needs_layout_passes=False),
)
def _sc_edges(q_hbm, k_hbm, v_hbm, c1_hbm, c2_hbm, upd_hbm,
              c1b, c2b, qkv, sems):
    w = lax.axis_index("s") * NC + lax.axis_index("c")
    pltpu.sync_copy(c1_hbm.at[pl.ds(w * E_PW, E_PW)], c1b)
    pltpu.sync_copy(c2_hbm.at[pl.ds(w * E_PW, E_PW)], c2b)
    nd = OUT_DIM // 16
    slots = [(qkv[0], qkv[1], qkv[2], sems[0], sems[1], sems[2]),
             (qkv[3], qkv[4], qkv[5], sems[3], sems[4], sems[5])]

    def fire(i, slot):
        off, sz = E_CHUNKS[i]
        qr, kr, vr, sq, sk, sv_ = slot
        return [
            pltpu.async_copy(q_hbm.at[c1b.at[pl.ds(off, sz)]],
                             qr.at[pl.ds(0, sz)], sq),
            pltpu.async_copy(k_hbm.at[c2b.at[pl.ds(off, sz)]],
                             kr.at[pl.ds(0, sz)], sk),
            pltpu.async_copy(v_hbm.at[c2b.at[pl.ds(off, sz)]],
                             vr.at[pl.ds(0, sz)], sv_),
        ]

    pend = {0: fire(0, slots[0])}
    for i in range(len(E_CHUNKS)):
        off, sz = E_CHUNKS[i]
        qr, kr, vr = slots[i % 2][:3]
        if i + 1 < len(E_CHUNKS):
            pend[i + 1] = fire(i + 1, slots[(i + 1) % 2])
        for cp in pend.pop(i):
            cp.wait()

        def edge_body(e, _):
            acc = qr[e, pl.ds(0, 16)] * kr[e, pl.ds(0, 16)]
            for d in range(1, nd):
                acc = acc + qr[e, pl.ds(d * 16, 16)] * kr[e, pl.ds(d * 16, 16)]
            sv = _allreduce(acc, jnp.add)
            sv = jnp.maximum(sv, jnp.float32(NEG_SLOPE) * sv)
            aw = jnp.float32(1.0) / (jnp.float32(1.0) + jnp.exp(-sv))
            for d in range(nd):
                vr[e, pl.ds(d * 16, 16)] = vr[e, pl.ds(d * 16, 16)] * aw
            return 0

        lax.fori_loop(0, sz, edge_body, 0)
        pltpu.sync_copy(vr.at[pl.ds(0, sz)],
                        upd_hbm.at[pl.ds(w * E_PW + off, sz)])


CAP = 16384
DEPTH4 = 4


@functools.partial(
    pl.kernel,
    out_type=jax.ShapeDtypeStruct((NUM_VARS, OUT_DIM), jnp.float32),
    mesh=_mesh,
    scratch_types=[
        pltpu.VMEM((S_PT,), jnp.int32),
        pltpu.VMEM((CAP,), jnp.int32),
        [pltpu.VMEM((1, BATCH), jnp.int32) for _ in range(DEPTH4)],
        [pltpu.VMEM((BATCH,), jnp.int32) for _ in range(DEPTH4)],
        [pltpu.VMEM((BATCH, OUT_DIM), jnp.float32) for _ in range(DEPTH4)],
        pltpu.VMEM_SHARED((CH + 8, OUT_DIM), jnp.float32),
        [pltpu.SemaphoreType.DMA for _ in range(DEPTH4)],
        [pltpu.SemaphoreType.DMA for _ in range(DEPTH4)],
    ],
    compiler_params=pltpu.CompilerParams(needs_layout_passes=False),
)
def _sc_scatter(x_hbm, sv_hbm, upd_hbm, out_hbm,
                svb, colbuf, idxs, edgs, rows, chunk, gsems, ssems):
    c = lax.axis_index("c")
    s = lax.axis_index("s")
    pltpu.sync_copy(sv_hbm.at[pl.ds(s * S_PT, S_PT)], svb)
    slot0 = s * S_PT
    lane = lax.iota(jnp.int32, 16)

    def fire_gather(b, j):
        for v in range(RPB):
            code = colbuf[pl.ds(b * BATCH + v * 16, 16)]
            idxs[j][0, pl.ds(v * 16, 16)] = code & jnp.int32(16383)
            edgs[j][pl.ds(v * 16, 16)] = code >> 14
        pltpu.async_copy(upd_hbm.at[edgs[j]], rows[j], gsems[j])

    def wait_gather(j):
        pltpu.make_async_copy(upd_hbm.at[edgs[j]], rows[j], gsems[j]).wait()

    for k in range(NCHUNK):
        base = (c * NCHUNK + k) * CH
        pltpu.sync_copy(x_hbm.at[pl.ds(base + s * R_PT, R_PT)],
                        chunk.at[pl.ds(s * R_PT, R_PT)])

        @pl.when(s == 0)
        def _():
            pltpu.sync_copy(x_hbm.at[pl.ds(base + NS * R_PT, R_TAIL)],
                            chunk.at[pl.ds(NS * R_PT, R_TAIL)])

        plsc.subcore_barrier()

        def scan_body(jv, curv):
            idx = svb[pl.ds(jv * 16, 16)]
            m = (idx >= base) & (idx < base + CH)
            ge = (slot0 + jv * 16 + lane) >> 3
            code = (idx - base) | (ge << 14)
            plsc.store_scatter(colbuf, [curv * 16 + lane], code, mask=m)
            return curv + m.astype(jnp.int32)

        curv = lax.fori_loop(0, S_PT // 16, scan_body,
                             jnp.zeros((16,), jnp.int32))
        maxl = _allreduce(curv, jnp.maximum)[0]
        minl = _allreduce(curv, jnp.minimum)[0]
        nb = (maxl + RPB - 1) >> 1
        nb4 = ((nb + DEPTH4 - 1) >> 2) << 2
        junk = jnp.full((16,), CH, jnp.int32)

        def fill_body(r, _):
            plsc.store_scatter(colbuf, [r * 16 + lane], junk, mask=(r >= curv))
            return 0

        lax.fori_loop(minl, (nb4 + DEPTH4) * RPB, fill_body, 0)

        for j in range(DEPTH4):
            fire_gather(j, j)

        def body(g, _):
            for j in range(DEPTH4):
                wait_gather(j)
                pltpu.async_copy(rows[j], chunk.at[idxs[j].at[0]], ssems[j],
                                 add=True)
            for j in range(DEPTH4):
                pltpu.make_async_copy(rows[j], chunk.at[idxs[j].at[0]],
                                      ssems[j]).wait()
                fire_gather(DEPTH4 * g + DEPTH4 + j, j)
            return 0

        lax.fori_loop(0, nb4 >> 2, body, 0)
        for j in range(DEPTH4):
            wait_gather(j)
        plsc.subcore_barrier()
        pltpu.sync_copy(chunk.at[pl.ds(s * R_PT, R_PT)],
                        out_hbm.at[pl.ds(base + s * R_PT, R_PT)])

        @pl.when(s == 0)
        def _():
            pltpu.sync_copy(chunk.at[pl.ds(NS * R_PT, R_TAIL)],
                            out_hbm.at[pl.ds(base + NS * R_PT, R_TAIL)])


def kernel(x_var, cluster_var_ids, cluster_edge_index, shared_vars,
           W_Q, W_K, W_V, head_weights):
    cvi = jnp.pad(cluster_var_ids,
                  ((0, C_PAD - NUM_CLUSTERS), (0, VPC_PAD - VARS_PER_CLUSTER)))
    cvi = cvi.reshape(-1)
    c1 = cluster_edge_index[0]
    c2 = cluster_edge_index[1]
    sv_flat = shared_vars.reshape(-1)
    hw_pad = jnp.zeros((8, 128), jnp.float32).at[0, :4].set(head_weights)

    cf = _sc_cluster_means(x_var, cvi)
    q, kk, v = _tc_qkv(cf, W_Q, W_K, W_V, hw_pad)
    upd = _sc_edges(q, kk, v, c1, c2)
    out = _sc_scatter(x_var, sv_flat, upd)
    return out


# speedup vs baseline: 6.9650x; 6.9650x over previous
import functools

import jax
import jax.numpy as jnp
from jax import lax
from jax.experimental import pallas as pl
from jax.experimental.pallas import tpu as pltpu
from jax.experimental.pallas import tpu_sc as plsc

NUM_VARS = 100000
IN_DIM = 128
OUT_DIM = 128
NUM_CLUSTERS = 2000
VARS_PER_CLUSTER = 50
NUM_CEDGES = 32000
SHARED_PER_EDGE = 8
NEG_SLOPE = 0.2

NC = 2
NS = 16
NW = NC * NS

C_PAD = 2048
CPW = C_PAD // NW
VPC_PAD = 56
IDX_ROW = CPW * VPC_PAD

E_PW = NUM_CEDGES // NW
E_CHUNKS = [(0, 128), (128, 128), (256, 128), (384, 128),
            (512, 128), (640, 128), (768, 128), (896, 104)]

S_PT = NUM_CEDGES // NS * SHARED_PER_EDGE
CH = 10000
NCHUNK = NUM_VARS // (NC * CH)
R_PT = 624
R_TAIL = CH - NS * R_PT
BATCH = 32
RPB = BATCH // 16

_mesh = plsc.VectorSubcoreMesh(
    core_axis_name="c", subcore_axis_name="s", num_cores=NC, num_subcores=NS)

_GDN = lax.GatherDimensionNumbers(
    offset_dims=(), collapsed_slice_dims=(0,), start_index_map=(0,))


def _rot(v, sh):
    idx = ((lax.iota(jnp.int32, 16) + sh) & 15)[:, None]
    return lax.gather(v, idx, _GDN, (1,),
                      mode=lax.GatherScatterMode.PROMISE_IN_BOUNDS)


def _allreduce(v, op):
    for sh in (8, 4, 2, 1):
        v = op(v, _rot(v, sh))
    return v


GCL = 8
G2 = GCL * VPC_PAD
NGRP = CPW // GCL
DEPTH1 = 2


@functools.partial(
    pl.kernel,
    out_type=jax.ShapeDtypeStruct((C_PAD, IN_DIM), jnp.float32),
    mesh=_mesh,
    scratch_types=[
        pltpu.VMEM((IDX_ROW + DEPTH1 * G2,), jnp.int32),
        [pltpu.VMEM((G2, IN_DIM), jnp.float32) for _ in range(DEPTH1)],
        pltpu.VMEM((GCL, IN_DIM), jnp.float32),
        [pltpu.SemaphoreType.DMA for _ in range(DEPTH1)],
    ],
    compiler_params=pltpu.CompilerParams(needs_layout_passes=False),
)
def _sc_cluster_means(x_hbm, cvi_hbm, cf_hbm, idxb, rows, cfb, sems):
    w = lax.axis_index("s") * NC + lax.axis_index("c")
    pltpu.sync_copy(cvi_hbm.at[pl.ds(w * IDX_ROW, IDX_ROW)],
                    idxb.at[pl.ds(0, IDX_ROW)])
    lane1 = lax.iota(jnp.int32, 16)
    for v in range(DEPTH1 * G2 // 16):
        idxb[pl.ds(IDX_ROW + v * 16, 16)] = (lane1 * 97 + v * 1553) & 65535

    def fire(p, j):
        pltpu.async_copy(x_hbm.at[idxb.at[pl.ds(p * G2, G2)]], rows[j],
                         sems[j])

    def wait(j):
        pltpu.make_async_copy(x_hbm.at[idxb.at[pl.ds(0, G2)]], rows[j],
                              sems[j]).wait()

    def reduce_group(p, j):
        inv = jnp.float32(1.0 / VARS_PER_CLUSTER)
        for i in range(GCL):
            def racc(r, accs):
                return tuple(
                    accs[d] + rows[j][i * VPC_PAD + r, pl.ds(d * 16, 16)]
                    for d in range(IN_DIM // 16))
            accs = lax.fori_loop(
                0, VARS_PER_CLUSTER, racc,
                tuple(jnp.zeros((16,), jnp.float32)
                      for _ in range(IN_DIM // 16)))
            for d in range(IN_DIM // 16):
                cfb[i, pl.ds(d * 16, 16)] = accs[d] * inv

    for j in range(DEPTH1):
        fire(j, j)

    def body(g, _):
        for j in range(DEPTH1):
            p = DEPTH1 * g + j
            wait(j)
            reduce_group(p, j)
            fire(p + DEPTH1, j)
            pltpu.sync_copy(cfb, cf_hbm.at[pl.ds(w * CPW + p * GCL, GCL)])
        return 0

    lax.fori_loop(0, NGRP // DEPTH1, body, 0)
    for j in range(DEPTH1):
        wait(j)


def _tc_qkv_body(cf_ref, wq_ref, wk_ref, wv_ref, hw_ref, q_ref, k_ref, v_ref):
    cf = cf_ref[...]
    dn = (((1,), (1,)), ((), ()))
    scale = jnp.float32(1.0) / jnp.sqrt(jnp.float32(OUT_DIM))
    q_ref[...] = lax.dot_general(
        cf, wq_ref[...], dn, preferred_element_type=jnp.float32) * scale
    k_ref[...] = lax.dot_general(
        cf, wk_ref[...], dn, preferred_element_type=jnp.float32)
    hw = jnp.sum(hw_ref[...]) * jnp.float32(1.0 / 4.0)
    v_ref[...] = lax.dot_general(
        cf, wv_ref[...], dn, preferred_element_type=jnp.float32) * hw


_tc_qkv = pl.pallas_call(
    _tc_qkv_body,
    out_shape=[jax.ShapeDtypeStruct((C_PAD, OUT_DIM), jnp.float32)] * 3,
)


@functools.partial(
    pl.kernel,
    out_type=jax.ShapeDtypeStruct((NUM_CEDGES, OUT_DIM), jnp.float32),
    mesh=_mesh,
    scratch_types=[
        pltpu.VMEM((E_PW,), jnp.int32),
        pltpu.VMEM((E_PW,), jnp.int32),
        [pltpu.VMEM((128, OUT_DIM), jnp.float32) for _ in range(6)],
        [pltpu.SemaphoreType.DMA for _ in range(6)],
    ],
    compiler_params=pltpu.CompilerParams(needs_layout_passes=False),
)
def _sc_edges(q_hbm, k_hbm, v_hbm, c1_hbm, c2_hbm, upd_hbm,
              c1b, c2b, qkv, sems):
    w = lax.axis_index("s") * NC + lax.axis_index("c")
    pltpu.sync_copy(c1_hbm.at[pl.ds(w * E_PW, E_PW)], c1b)
    pltpu.sync_copy(c2_hbm.at[pl.ds(w * E_PW, E_PW)], c2b)
    nd = OUT_DIM // 16
    slots = [(qkv[0], qkv[1], qkv[2], sems[0], sems[1], sems[2]),
             (qkv[3], qkv[4], qkv[5], sems[3], sems[4], sems[5])]

    def fire(i, slot):
        off, sz = E_CHUNKS[i]
        qr, kr, vr, sq, sk, sv_ = slot
        return [
            pltpu.async_copy(q_hbm.at[c1b.at[pl.ds(off, sz)]],
                             qr.at[pl.ds(0, sz)], sq),
            pltpu.async_copy(k_hbm.at[c2b.at[pl.ds(off, sz)]],
                             kr.at[pl.ds(0, sz)], sk),
            pltpu.async_copy(v_hbm.at[c2b.at[pl.ds(off, sz)]],
                             vr.at[pl.ds(0, sz)], sv_),
        ]

    pend = {0: fire(0, slots[0])}
    for i in range(len(E_CHUNKS)):
        off, sz = E_CHUNKS[i]
        qr, kr, vr = slots[i % 2][:3]
        if i + 1 < len(E_CHUNKS):
            pend[i + 1] = fire(i + 1, slots[(i + 1) % 2])
        for cp in pend.pop(i):
            cp.wait()

        def edge_body(e, _):
            acc = qr[e, pl.ds(0, 16)] * kr[e, pl.ds(0, 16)]
            for d in range(1, nd):
                acc = acc + qr[e, pl.ds(d * 16, 16)] * kr[e, pl.ds(d * 16, 16)]
            sv = _allreduce(acc, jnp.add)
            sv = jnp.maximum(sv, jnp.float32(NEG_SLOPE) * sv)
            aw = jnp.float32(1.0) / (jnp.float32(1.0) + jnp.exp(-sv))
            for d in range(nd):
                vr[e, pl.ds(d * 16, 16)] = vr[e, pl.ds(d * 16, 16)] * aw
            return 0

        lax.fori_loop(0, sz, edge_body, 0)
        pltpu.sync_copy(vr.at[pl.ds(0, sz)],
                        upd_hbm.at[pl.ds(w * E_PW + off, sz)])


S_HALF = S_PT // 2
CAP = 9216
BATCH = 96
RPB = BATCH // 16
DEPTH4 = 2


@functools.partial(
    pl.kernel,
    out_type=jax.ShapeDtypeStruct((NUM_VARS, OUT_DIM), jnp.float32),
    mesh=_mesh,
    scratch_types=[
        pltpu.VMEM((S_PT,), jnp.int32),
        pltpu.VMEM((CAP,), jnp.int32),
        [pltpu.VMEM((1, BATCH), jnp.int32) for _ in range(DEPTH4)],
        [pltpu.VMEM((BATCH,), jnp.int32) for _ in range(DEPTH4)],
        [pltpu.VMEM((BATCH, OUT_DIM), jnp.float32) for _ in range(DEPTH4)],
        pltpu.VMEM_SHARED((CH + 16, OUT_DIM), jnp.float32),
        [pltpu.SemaphoreType.DMA for _ in range(DEPTH4)],
        [pltpu.SemaphoreType.DMA for _ in range(DEPTH4)],
    ],
    compiler_params=pltpu.CompilerParams(needs_layout_passes=False),
)
def _sc_scatter(x_hbm, sv_hbm, upd_hbm, out_hbm,
                svb, colbuf, idxs, edgs, rows, chunk, gsems, ssems):
    c = lax.axis_index("c")
    s = lax.axis_index("s")
    pltpu.sync_copy(sv_hbm.at[pl.ds(s * S_PT, S_PT)], svb)
    lane = lax.iota(jnp.int32, 16)
    junk = CH + lane + (lane << 14)

    def fire_gather(b, j):
        for v in range(RPB):
            code = colbuf[pl.ds(b * BATCH + v * 16, 16)]
            idxs[j][0, pl.ds(v * 16, 16)] = code & jnp.int32(16383)
            edgs[j][pl.ds(v * 16, 16)] = code >> 14
        pltpu.async_copy(upd_hbm.at[edgs[j]], rows[j], gsems[j])

    def wait_gather(j):
        pltpu.make_async_copy(upd_hbm.at[edgs[j]], rows[j], gsems[j]).wait()

    for k in range(NCHUNK):
        base = (c * NCHUNK + k) * CH
        pltpu.sync_copy(x_hbm.at[pl.ds(base + s * R_PT, R_PT)],
                        chunk.at[pl.ds(s * R_PT, R_PT)])

        @pl.when(s == 0)
        def _():
            pltpu.sync_copy(x_hbm.at[pl.ds(base + NS * R_PT, R_TAIL)],
                            chunk.at[pl.ds(NS * R_PT, R_TAIL)])

        plsc.subcore_barrier()

        for half in range(2):
            slot0 = s * S_PT + half * S_HALF

            def scan_body(jv, curv):
                idx = svb[pl.ds(half * S_HALF + jv * 16, 16)]
                m = (idx >= base) & (idx < base + CH)
                ge = (slot0 + jv * 16 + lane) >> 3
                code = (idx - base) | (ge << 14)
                plsc.store_scatter(colbuf, [curv * 16 + lane], code, mask=m)
                return curv + m.astype(jnp.int32)

            curv = lax.fori_loop(0, S_HALF // 16, scan_body,
                                 jnp.zeros((16,), jnp.int32))
            maxl = _allreduce(curv, jnp.maximum)[0]
            minl = _allreduce(curv, jnp.minimum)[0]
            nb = (maxl + RPB - 1) // RPB
            nb2 = ((nb + 1) >> 1) << 1

            def fill_body(r, _):
                plsc.store_scatter(colbuf, [r * 16 + lane], junk,
                                   mask=(r >= curv))
                return 0

            lax.fori_loop(minl, (nb2 + DEPTH4) * RPB, fill_body, 0)

            fire_gather(0, 0)
            fire_gather(1, 1)

            def body(g, _):
                for j in range(DEPTH4):
                    wait_gather(j)
                    cp = pltpu.async_copy(rows[j], chunk.at[idxs[j].at[0]],
                                          ssems[j], add=True)
                    cp.wait()
                    fire_gather(DEPTH4 * g + DEPTH4 + j, j)
                return 0

            lax.fori_loop(0, nb2 >> 1, body, 0)
            for j in range(DEPTH4):
                wait_gather(j)

        plsc.subcore_barrier()
        pltpu.sync_copy(chunk.at[pl.ds(s * R_PT, R_PT)],
                        out_hbm.at[pl.ds(base + s * R_PT, R_PT)])

        @pl.when(s == 0)
        def _():
            pltpu.sync_copy(chunk.at[pl.ds(NS * R_PT, R_TAIL)],
                            out_hbm.at[pl.ds(base + NS * R_PT, R_TAIL)])


def kernel(x_var, cluster_var_ids, cluster_edge_index, shared_vars,
           W_Q, W_K, W_V, head_weights):
    minor_pad = cluster_var_ids[:, :VPC_PAD - VARS_PER_CLUSTER]
    cvi_m = jnp.concatenate([cluster_var_ids, minor_pad], axis=1)
    row_pad = (jnp.arange((C_PAD - NUM_CLUSTERS) * VPC_PAD,
                          dtype=jnp.int32) * 2087) % NUM_VARS
    cvi = jnp.concatenate(
        [cvi_m, row_pad.reshape(C_PAD - NUM_CLUSTERS, VPC_PAD)],
        axis=0).reshape(-1)
    c1 = cluster_edge_index[0]
    c2 = cluster_edge_index[1]
    sv_flat = shared_vars.reshape(-1)
    hw_pad = jnp.zeros((8, 128), jnp.float32).at[0, :4].set(head_weights)

    cf = _sc_cluster_means(x_var, cvi)
    q, kk, v = _tc_qkv(cf, W_Q, W_K, W_V, hw_pad)
    upd = _sc_edges(q, kk, v, c1, c2)
    out = _sc_scatter(x_var, sv_flat, upd)
    return out


# speedup vs baseline: 9.1560x; 1.3146x over previous
import functools

import jax
import jax.numpy as jnp
from jax import lax
from jax.experimental import pallas as pl
from jax.experimental.pallas import tpu as pltpu
from jax.experimental.pallas import tpu_sc as plsc

NUM_VARS = 100000
IN_DIM = 128
OUT_DIM = 128
NUM_CLUSTERS = 2000
VARS_PER_CLUSTER = 50
NUM_CEDGES = 32000
SHARED_PER_EDGE = 8
NEG_SLOPE = 0.2

NC = 2
NS = 16
NW = NC * NS

C_PAD = 2048
CPW = C_PAD // NW
VPC_PAD = 56
IDX_ROW = CPW * VPC_PAD

E_PW = NUM_CEDGES // NW
E_CHUNKS = [(0, 128), (128, 128), (256, 128), (384, 128),
            (512, 128), (640, 128), (768, 128), (896, 104)]

S_PT = NUM_CEDGES // NS * SHARED_PER_EDGE
CH = 10000
NCHUNK = NUM_VARS // (NC * CH)
R_PT = 624
R_TAIL = CH - NS * R_PT
BATCH = 32
RPB = BATCH // 16

_mesh = plsc.VectorSubcoreMesh(
    core_axis_name="c", subcore_axis_name="s", num_cores=NC, num_subcores=NS)

_GDN = lax.GatherDimensionNumbers(
    offset_dims=(), collapsed_slice_dims=(0,), start_index_map=(0,))


def _rot(v, sh):
    idx = ((lax.iota(jnp.int32, 16) + sh) & 15)[:, None]
    return lax.gather(v, idx, _GDN, (1,),
                      mode=lax.GatherScatterMode.PROMISE_IN_BOUNDS)


def _allreduce(v, op):
    for sh in (8, 4, 2, 1):
        v = op(v, _rot(v, sh))
    return v


GCL = 8
G2 = GCL * VPC_PAD
NGRP = CPW // GCL
DEPTH1 = 2


@functools.partial(
    pl.kernel,
    out_type=jax.ShapeDtypeStruct((C_PAD, IN_DIM), jnp.float32),
    mesh=_mesh,
    scratch_types=[
        pltpu.VMEM((IDX_ROW + DEPTH1 * G2,), jnp.int32),
        [pltpu.VMEM((G2, IN_DIM), jnp.float32) for _ in range(DEPTH1)],
        pltpu.VMEM((GCL, IN_DIM), jnp.float32),
        [pltpu.SemaphoreType.DMA for _ in range(DEPTH1)],
    ],
    compiler_params=pltpu.CompilerParams(needs_layout_passes=False),
)
def _sc_cluster_means(x_hbm, cvi_hbm, cf_hbm, idxb, rows, cfb, sems):
    w = lax.axis_index("s") * NC + lax.axis_index("c")
    pltpu.sync_copy(cvi_hbm.at[pl.ds(w * IDX_ROW, IDX_ROW)],
                    idxb.at[pl.ds(0, IDX_ROW)])
    lane1 = lax.iota(jnp.int32, 16)
    for v in range(DEPTH1 * G2 // 16):
        idxb[pl.ds(IDX_ROW + v * 16, 16)] = (lane1 * 97 + v * 1553) & 65535

    def fire(p, j):
        pltpu.async_copy(x_hbm.at[idxb.at[pl.ds(p * G2, G2)]], rows[j],
                         sems[j])

    def wait(j):
        pltpu.make_async_copy(x_hbm.at[idxb.at[pl.ds(0, G2)]], rows[j],
                              sems[j]).wait()

    def reduce_group(p, j):
        inv = jnp.float32(1.0 / VARS_PER_CLUSTER)
        for i in range(GCL):
            def racc(r, accs):
                return tuple(
                    accs[d] + rows[j][i * VPC_PAD + r, pl.ds(d * 16, 16)]
                    for d in range(IN_DIM // 16))
            accs = lax.fori_loop(
                0, VARS_PER_CLUSTER, racc,
                tuple(jnp.zeros((16,), jnp.float32)
                      for _ in range(IN_DIM // 16)))
            for d in range(IN_DIM // 16):
                cfb[i, pl.ds(d * 16, 16)] = accs[d] * inv

    for j in range(DEPTH1):
        fire(j, j)

    def body(g, _):
        for j in range(DEPTH1):
            p = DEPTH1 * g + j
            wait(j)
            reduce_group(p, j)
            fire(p + DEPTH1, j)
            pltpu.sync_copy(cfb, cf_hbm.at[pl.ds(w * CPW + p * GCL, GCL)])
        return 0

    lax.fori_loop(0, NGRP // DEPTH1, body, 0)
    for j in range(DEPTH1):
        wait(j)


def _tc_qkv_body(cf_ref, wq_ref, wk_ref, wv_ref, hw_ref, q_ref, k_ref, v_ref):
    cf = cf_ref[...]
    dn = (((1,), (1,)), ((), ()))
    scale = jnp.float32(1.0) / jnp.sqrt(jnp.float32(OUT_DIM))
    q_ref[...] = lax.dot_general(
        cf, wq_ref[...], dn, preferred_element_type=jnp.float32) * scale
    k_ref[...] = lax.dot_general(
        cf, wk_ref[...], dn, preferred_element_type=jnp.float32)
    hw = jnp.sum(hw_ref[...]) * jnp.float32(1.0 / 4.0)
    v_ref[...] = lax.dot_general(
        cf, wv_ref[...], dn, preferred_element_type=jnp.float32) * hw


_tc_qkv = pl.pallas_call(
    _tc_qkv_body,
    out_shape=[jax.ShapeDtypeStruct((C_PAD, OUT_DIM), jnp.float32)] * 3,
)


@functools.partial(
    pl.kernel,
    out_type=jax.ShapeDtypeStruct((NUM_CEDGES, OUT_DIM), jnp.float32),
    mesh=_mesh,
    scratch_types=[
        pltpu.VMEM((E_PW,), jnp.int32),
        pltpu.VMEM((E_PW,), jnp.int32),
        [pltpu.VMEM((128, OUT_DIM), jnp.float32) for _ in range(6)],
        [pltpu.SemaphoreType.DMA for _ in range(6)],
    ],
    compiler_params=pltpu.CompilerParams(needs_layout_passes=False),
)
def _sc_edges(q_hbm, k_hbm, v_hbm, c1_hbm, c2_hbm, upd_hbm,
              c1b, c2b, qkv, sems):
    w = lax.axis_index("s") * NC + lax.axis_index("c")
    pltpu.sync_copy(c1_hbm.at[pl.ds(w * E_PW, E_PW)], c1b)
    pltpu.sync_copy(c2_hbm.at[pl.ds(w * E_PW, E_PW)], c2b)
    nd = OUT_DIM // 16
    slots = [(qkv[0], qkv[1], qkv[2], sems[0], sems[1], sems[2]),
             (qkv[3], qkv[4], qkv[5], sems[3], sems[4], sems[5])]

    def fire(i, slot):
        off, sz = E_CHUNKS[i]
        qr, kr, vr, sq, sk, sv_ = slot
        return [
            pltpu.async_copy(q_hbm.at[c1b.at[pl.ds(off, sz)]],
                             qr.at[pl.ds(0, sz)], sq),
            pltpu.async_copy(k_hbm.at[c2b.at[pl.ds(off, sz)]],
                             kr.at[pl.ds(0, sz)], sk),
            pltpu.async_copy(v_hbm.at[c2b.at[pl.ds(off, sz)]],
                             vr.at[pl.ds(0, sz)], sv_),
        ]

    pend = {0: fire(0, slots[0])}
    for i in range(len(E_CHUNKS)):
        off, sz = E_CHUNKS[i]
        qr, kr, vr = slots[i % 2][:3]
        if i + 1 < len(E_CHUNKS):
            pend[i + 1] = fire(i + 1, slots[(i + 1) % 2])
        for cp in pend.pop(i):
            cp.wait()

        def edge_body(e, _):
            acc = qr[e, pl.ds(0, 16)] * kr[e, pl.ds(0, 16)]
            for d in range(1, nd):
                acc = acc + qr[e, pl.ds(d * 16, 16)] * kr[e, pl.ds(d * 16, 16)]
            sv = _allreduce(acc, jnp.add)
            sv = jnp.maximum(sv, jnp.float32(NEG_SLOPE) * sv)
            aw = jnp.float32(1.0) / (jnp.float32(1.0) + jnp.exp(-sv))
            for d in range(nd):
                vr[e, pl.ds(d * 16, 16)] = vr[e, pl.ds(d * 16, 16)] * aw
            return 0

        lax.fori_loop(0, sz, edge_body, 0)
        pltpu.sync_copy(vr.at[pl.ds(0, sz)],
                        upd_hbm.at[pl.ds(w * E_PW + off, sz)])


S_HALF = S_PT // 2
CAP = 9216
BATCH = 96
RPB = BATCH // 16
DEPTH4 = 2


@functools.partial(
    pl.kernel,
    out_type=jax.ShapeDtypeStruct((NUM_VARS, OUT_DIM), jnp.float32),
    mesh=_mesh,
    scratch_types=[
        pltpu.VMEM((S_PT,), jnp.int32),
        pltpu.VMEM((CAP,), jnp.int32),
        [pltpu.VMEM((1, BATCH), jnp.int32) for _ in range(DEPTH4)],
        [pltpu.VMEM((BATCH,), jnp.int32) for _ in range(DEPTH4)],
        [pltpu.VMEM((BATCH, OUT_DIM), jnp.float32) for _ in range(DEPTH4)],
        pltpu.VMEM_SHARED((CH + 16, OUT_DIM), jnp.float32),
        [pltpu.SemaphoreType.DMA for _ in range(DEPTH4)],
        [pltpu.SemaphoreType.DMA for _ in range(DEPTH4)],
    ],
    compiler_params=pltpu.CompilerParams(needs_layout_passes=False),
)
def _sc_scatter(x_hbm, sv_hbm, upd_hbm, out_hbm,
                svb, colbuf, idxs, edgs, rows, chunk, gsems, ssems):
    c = lax.axis_index("c")
    s = lax.axis_index("s")
    pltpu.sync_copy(sv_hbm.at[pl.ds(s * S_PT, S_PT)], svb)
    lane = lax.iota(jnp.int32, 16)
    junk = CH + lane + (lane << 14)

    def fire_gather(b, j):
        for v in range(RPB):
            code = colbuf[pl.ds(b * BATCH + v * 16, 16)]
            idxs[j][0, pl.ds(v * 16, 16)] = code & jnp.int32(16383)
            edgs[j][pl.ds(v * 16, 16)] = code >> 14
        pltpu.async_copy(upd_hbm.at[edgs[j]], rows[j], gsems[j])

    def wait_gather(j):
        pltpu.make_async_copy(upd_hbm.at[edgs[j]], rows[j], gsems[j]).wait()

    for k in range(NCHUNK):
        base = (c * NCHUNK + k) * CH
        pltpu.sync_copy(x_hbm.at[pl.ds(base + s * R_PT, R_PT)],
                        chunk.at[pl.ds(s * R_PT, R_PT)])

        @pl.when(s == 0)
        def _():
            pltpu.sync_copy(x_hbm.at[pl.ds(base + NS * R_PT, R_TAIL)],
                            chunk.at[pl.ds(NS * R_PT, R_TAIL)])

        plsc.subcore_barrier()

        for half in range(2):
            slot0 = s * S_PT + half * S_HALF

            def scan_body(jv, curv):
                idx = svb[pl.ds(half * S_HALF + jv * 16, 16)]
                m = (idx >= base) & (idx < base + CH)
                ge = (slot0 + jv * 16 + lane) >> 3
                code = (idx - base) | (ge << 14)
                plsc.store_scatter(colbuf, [curv * 16 + lane], code, mask=m)
                return curv + m.astype(jnp.int32)

            curv = lax.fori_loop(0, S_HALF // 16, scan_body,
                                 jnp.zeros((16,), jnp.int32))
            maxl = _allreduce(curv, jnp.maximum)[0]
            minl = _allreduce(curv, jnp.minimum)[0]
            nb = (maxl + RPB - 1) // RPB
            nb2 = ((nb + 1) >> 1) << 1

            def fill_body(r, _):
                plsc.store_scatter(colbuf, [r * 16 + lane], junk,
                                   mask=(r >= curv))
                return 0

            lax.fori_loop(minl, nb * RPB, fill_body, 0)

            for j in range(DEPTH4):
                @pl.when(j < nb)
                def _(j=j):
                    fire_gather(j, j)

            def body(g, _):
                for j in range(DEPTH4):
                    b = DEPTH4 * g + j

                    @pl.when(b < nb)
                    def _(b=b, j=j):
                        wait_gather(j)
                        cp = pltpu.async_copy(rows[j],
                                              chunk.at[idxs[j].at[0]],
                                              ssems[j], add=True)
                        cp.wait()

                        @pl.when(b + DEPTH4 < nb)
                        def _():
                            fire_gather(b + DEPTH4, j)
                return 0

            lax.fori_loop(0, nb2 >> 1, body, 0)

        plsc.subcore_barrier()
        pltpu.sync_copy(chunk.at[pl.ds(s * R_PT, R_PT)],
                        out_hbm.at[pl.ds(base + s * R_PT, R_PT)])

        @pl.when(s == 0)
        def _():
            pltpu.sync_copy(chunk.at[pl.ds(NS * R_PT, R_TAIL)],
                            out_hbm.at[pl.ds(base + NS * R_PT, R_TAIL)])


def kernel(x_var, cluster_var_ids, cluster_edge_index, shared_vars,
           W_Q, W_K, W_V, head_weights):
    minor_pad = cluster_var_ids[:, :VPC_PAD - VARS_PER_CLUSTER]
    cvi_m = jnp.concatenate([cluster_var_ids, minor_pad], axis=1)
    row_pad = (jnp.arange((C_PAD - NUM_CLUSTERS) * VPC_PAD,
                          dtype=jnp.int32) * 2087) % NUM_VARS
    cvi = jnp.concatenate(
        [cvi_m, row_pad.reshape(C_PAD - NUM_CLUSTERS, VPC_PAD)],
        axis=0).reshape(-1)
    c1 = cluster_edge_index[0]
    c2 = cluster_edge_index[1]
    sv_flat = shared_vars.reshape(-1)
    hw_pad = jnp.zeros((8, 128), jnp.float32).at[0, :4].set(head_weights)

    cf = _sc_cluster_means(x_var, cvi)
    q, kk, v = _tc_qkv(cf, W_Q, W_K, W_V, hw_pad)
    upd = _sc_edges(q, kk, v, c1, c2)
    out = _sc_scatter(x_var, sv_flat, upd)
    return out
